# Initial kernel scaffold; baseline (speedup 1.0000x reference)
import jax
import jax.numpy as jnp
from jax.experimental import pallas as pl


def kernel(motion_input, codebook):
    raise NotImplementedError("write your pallas kernel here")


# speedup vs baseline: 1.8949x; 1.8949x over previous
import functools

import jax
import jax.numpy as jnp
from jax.experimental import pallas as pl

B, T, D, K = 16, 2048, 128, 1024
N = B * T
BM = 1024
NB = N // BM


def _vq_body(x_ref, cb_ref, idx_ref, q_ref, loss_ref):
    i = pl.program_id(0)
    x = x_ref[:]
    c = cb_ref[:]
    ab = jax.lax.dot_general(
        x, c, (((1,), (1,)), ((), ())),
        preferred_element_type=jnp.float32)
    xn = jnp.sum(x * x, axis=1, keepdims=True)
    cn = jnp.sum(c * c, axis=1)
    dist = -((xn - 2.0 * ab) + cn[None, :])
    idx = jnp.argmax(dist, axis=1).astype(jnp.int32)
    maxv = jnp.max(dist, axis=1)
    idx_ref[...] = idx.reshape(1, 1, BM)
    oh = (jax.lax.broadcasted_iota(jnp.int32, (BM, K), 1)
          == idx[:, None]).astype(jnp.float32)
    q_ref[:] = jax.lax.dot_general(
        oh, c, (((1,), (0,)), ((), ())),
        preferred_element_type=jnp.float32)
    partial = jnp.reshape(-jnp.sum(maxv), (1, 1))
    prev = jnp.where(i == 0, jnp.zeros((1, 1), jnp.float32), loss_ref[...])
    loss_ref[...] = prev + partial


@jax.jit
def _vq_tc(flat, codebook):
    return pl.pallas_call(
        _vq_body,
        grid=(NB,),
        in_specs=[
            pl.BlockSpec((BM, D), lambda i: (i, 0)),
            pl.BlockSpec((K, D), lambda i: (0, 0)),
        ],
        out_specs=[
            pl.BlockSpec((1, 1, BM), lambda i: (i, 0, 0)),
            pl.BlockSpec((BM, D), lambda i: (i, 0)),
            pl.BlockSpec((1, 1), lambda i: (0, 0)),
        ],
        out_shape=[
            jax.ShapeDtypeStruct((NB, 1, BM), jnp.int32),
            jax.ShapeDtypeStruct((N, D), jnp.float32),
            jax.ShapeDtypeStruct((1, 1), jnp.float32),
        ],
    )(flat, codebook)


def kernel(motion_input, codebook):
    flat = motion_input.reshape(N, D)
    idx3, q, loss_sum = _vq_tc(flat, codebook)
    embed_ind = idx3.reshape(B, T)
    quantize = q.reshape(B, T, D)
    loss = loss_sum[0, 0] / jnp.float32(N * D)
    return (quantize, embed_ind, loss)
